# baseline (speedup 1.0000x reference)
import jax
import jax.numpy as jnp
from jax.experimental import pallas as pl
from jax.experimental.pallas import tpu as pltpu

_TILE = 256
_K = 8


def _chamfer_kernel(x_ref, o_ref):
    n = x_ref.shape[1]
    p = jnp.swapaxes(x_ref[0], 0, 1)
    px = p[0:1, :]
    py = p[1:2, :]
    pz = p[2:3, :]
    pn = px * px + py * py + pz * pz
    hi = pn.astype(jnp.bfloat16)
    lo = (pn - hi.astype(jnp.float32)).astype(jnp.bfloat16)
    one = jnp.ones((1, n), jnp.bfloat16)
    zero = jnp.zeros((1, n), jnp.bfloat16)
    qb = jnp.concatenate(
        [px.astype(jnp.bfloat16), py.astype(jnp.bfloat16),
         pz.astype(jnp.bfloat16), one, one, hi, lo, zero], axis=0)
    pb = jnp.concatenate(
        [(2.0 * px).astype(jnp.bfloat16), (-2.0 * py).astype(jnp.bfloat16),
         (-2.0 * pz).astype(jnp.bfloat16), hi, lo, one, one, zero],
        axis=0)

    nt = n // _TILE
    acc = jnp.float32(0.0)
    run_col = jnp.full((n,), jnp.inf, jnp.float32)
    for ti in range(nt):
        base = ti * _TILE
        d = jax.lax.dot_general(
            qb[:, base:base + _TILE], pb[:, base:],
            (((0,), (0,)), ((), ())),
            preferred_element_type=jnp.float32)
        row_min = jnp.min(d, axis=1)
        dist = jnp.minimum(row_min, run_col[base:base + _TILE])
        acc = acc + jnp.sum(dist)
        if ti + 1 < nt:
            col_min = jnp.min(d, axis=0)
            upd = jnp.minimum(run_col[base:], col_min)
            run_col = upd if base == 0 else jnp.concatenate(
                [run_col[:base], upd])
    o_ref[0] = jnp.full((8, 128), acc, jnp.float32)


def kernel(xyz):
    B, N, _ = xyz.shape
    out = pl.pallas_call(
        _chamfer_kernel,
        grid=(B,),
        in_specs=[
            pl.BlockSpec((1, N, 3), lambda b: (b, 0, 0)),
        ],
        out_specs=pl.BlockSpec((1, 8, 128), lambda b: (b, 0, 0)),
        out_shape=jax.ShapeDtypeStruct((B, 8, 128), jnp.float32),
        compiler_params=pltpu.CompilerParams(
            dimension_semantics=("parallel",),
        ),
    )(xyz)
    return (2.0 / (B * N)) * jnp.sum(out[:, 0, 0])


# speedup vs baseline: 1.3062x; 1.3062x over previous
import jax
import jax.numpy as jnp
from jax.experimental import pallas as pl
from jax.experimental.pallas import tpu as pltpu

_TILE = 256
_K = 8


def _chamfer_kernel(p_ref, o_ref):
    n = p_ref.shape[2]
    p = p_ref[0]
    px = p[0:1, :]
    py = p[1:2, :]
    pz = p[2:3, :]
    pn = px * px + py * py + pz * pz
    hi = pn.astype(jnp.bfloat16)
    lo = (pn - hi.astype(jnp.float32)).astype(jnp.bfloat16)
    one = jnp.ones((1, n), jnp.bfloat16)
    zero = jnp.zeros((1, n), jnp.bfloat16)
    qb = jnp.concatenate(
        [px.astype(jnp.bfloat16), py.astype(jnp.bfloat16),
         pz.astype(jnp.bfloat16), one, one, hi, lo, zero], axis=0)
    pb = jnp.concatenate(
        [(2.0 * px).astype(jnp.bfloat16), (-2.0 * py).astype(jnp.bfloat16),
         (-2.0 * pz).astype(jnp.bfloat16), hi, lo, one, one, zero],
        axis=0)

    nt = n // _TILE
    acc = jnp.float32(0.0)
    run_col = jnp.full((n,), jnp.inf, jnp.float32)
    for ti in range(nt):
        base = ti * _TILE
        d = jax.lax.dot_general(
            qb[:, base:base + _TILE], pb[:, base:],
            (((0,), (0,)), ((), ())),
            preferred_element_type=jnp.float32)
        row_min = jnp.min(d, axis=1)
        dist = jnp.minimum(row_min, run_col[base:base + _TILE])
        acc = acc + jnp.sum(dist)
        if ti + 1 < nt:
            col_min = jnp.min(d, axis=0)
            upd = jnp.minimum(run_col[base:], col_min)
            run_col = upd if base == 0 else jnp.concatenate(
                [run_col[:base], upd])
    o_ref[0] = jnp.full((8, 128), acc, jnp.float32)


def kernel(xyz):
    B, N, _ = xyz.shape
    pmat = jnp.swapaxes(xyz, 1, 2)
    out = pl.pallas_call(
        _chamfer_kernel,
        grid=(B,),
        in_specs=[
            pl.BlockSpec((1, 3, N), lambda b: (b, 0, 0)),
        ],
        out_specs=pl.BlockSpec((1, 8, 128), lambda b: (b, 0, 0)),
        out_shape=jax.ShapeDtypeStruct((B, 8, 128), jnp.float32),
        compiler_params=pltpu.CompilerParams(
            dimension_semantics=("parallel",),
        ),
    )(pmat)
    return (2.0 / (B * N)) * jnp.sum(out[:, 0, 0])
